# baseline (speedup 1.0000x reference)
import functools

import jax
import jax.numpy as jnp
from jax import lax
from jax.experimental import pallas as pl
from jax.experimental.pallas import tpu as pltpu
from jax.experimental.pallas import tpu_sc as plsc

NU = 10000
NI = 40000
E = 500000
EPAD = 524288
PER_TILE = EPAD // 16
NSB = 16
S = PER_TILE // NSB
CH = 5000
SPM_ROWS = CH + 8
TPR = 312
SENT = 1 << 20
MAX_LOGVAR = 10.0


def _lrelu(x):
    return jnp.where(x >= 0, x, 0.01 * x)


def _make_agg(W):
    mesh = plsc.VectorSubcoreMesh(core_axis_name="c", subcore_axis_name="s")

    def body(xu_hbm, xi_hbm, src_hbm, dloc_hbm, aggu_hbm, aggi_hbm,
             src_v, dloc_v, gidx, sidx, rowbuf, zbuf, spm, gsem):
        core = lax.axis_index("c")
        sub = lax.axis_index("s")
        ebase = sub * PER_TILE
        start = pl.multiple_of(sub * TPR, 8)

        def zrow(r, _):
            def zcol(cc, _):
                zbuf[r, pl.ds(cc * 16, 16)] = jnp.zeros((16,), jnp.float32)
                return 0
            return lax.fori_loop(0, W // 16, zcol, 0)
        lax.fori_loop(0, 64, zrow, 0)

        spans = ((0, 64), (64, 64), (128, 64), (192, 64), (256, 64))

        def do_chunk(lo, scat_ref, gath_ref, tab_hbm, out_hbm):
            for off, sz in spans:
                pltpu.sync_copy(zbuf.at[pl.ds(0, sz)],
                                spm.at[pl.ds(start + off, sz)])
            plsc.subcore_barrier()

            def do_sb(sb, _):
                eoff = ebase + sb * S
                pltpu.sync_copy(src_hbm.at[pl.ds(eoff, S)], src_v)
                pltpu.sync_copy(dloc_hbm.at[pl.ds(eoff, S)], dloc_v)

                def scan_v(v, n):
                    sc = scat_ref[pl.ds(v * 16, 16)] - lo
                    g = gath_ref[pl.ds(v * 16, 16)]
                    m = (sc >= 0) & (sc < CH)
                    plsc.store_compressed(sidx.at[pl.ds(n, 16)], sc, mask=m)
                    plsc.store_compressed(gidx.at[pl.ds(n, 16)], g, mask=m)
                    return n + jnp.sum(m.astype(jnp.int32))
                n = lax.fori_loop(0, S // 16, scan_v, 0)

                sidx[pl.ds(n, 16)] = jnp.full((16,), CH, jnp.int32)
                for j in range(8):
                    gidx[pl.ds(n + j * 16, 16)] = jnp.zeros((16,), jnp.int32)

                def fire(k, _):
                    @pl.when(k * 128 < n)
                    def _():
                        pltpu.async_copy(
                            tab_hbm.at[gidx.at[pl.ds(k * 128, 128)]],
                            rowbuf, gsem).wait()
                        for j in range(8):
                            @pl.when(k * 128 + j * 16 < n)
                            def _():
                                sv = sidx[pl.ds(k * 128 + j * 16, 16)]
                                pltpu.sync_copy(
                                    rowbuf.at[pl.ds(j * 16, 16)],
                                    spm.at[sv], add=True)
                    return 0
                lax.fori_loop(0, S // 128, fire, 0)
                return 0
            lax.fori_loop(0, NSB, do_sb, 0)
            plsc.subcore_barrier()

            for off, sz in spans:
                pltpu.sync_copy(spm.at[pl.ds(start + off, sz)],
                                out_hbm.at[pl.ds(lo + start + off, sz)])
            plsc.subcore_barrier()

        do_chunk(core * CH, src_v, dloc_v, xi_hbm, aggu_hbm)
        for i in range(4):
            do_chunk((core + 2 * i) * CH, dloc_v, src_v, xu_hbm, aggi_hbm)

    kern = pl.kernel(
        body,
        out_type=(jax.ShapeDtypeStruct((NU, W), jnp.float32),
                  jax.ShapeDtypeStruct((NI, W), jnp.float32)),
        mesh=mesh,
        scratch_types=[
            pltpu.VMEM((S,), jnp.int32),
            pltpu.VMEM((S,), jnp.int32),
            pltpu.VMEM((S + 128,), jnp.int32),
            pltpu.VMEM((S + 16,), jnp.int32),
            pltpu.VMEM((128, W), jnp.float32),
            pltpu.VMEM((64, W), jnp.float32),
            pltpu.VMEM_SHARED((SPM_ROWS, W), jnp.float32),
            pltpu.SemaphoreType.DMA,
        ],
        compiler_params=pltpu.CompilerParams(needs_layout_passes=False,
                                             use_tc_tiling_on_sc=False),
    )
    return kern


_agg192 = _make_agg(192)


def _full(shape):
    return pl.BlockSpec(shape, lambda i: tuple(0 for _ in shape))


def _rows(bshape):
    return pl.BlockSpec(bshape, lambda i: (i,) + tuple(0 for _ in bshape[1:]))


def _normalize(y):
    nrm = jnp.maximum(jnp.sqrt(jnp.sum(y * y, axis=1, keepdims=True)), 1e-12)
    return y / nrm


def _split_write(out_a, out_b, m, y):
    if m == 0:
        out_a[:, 0:128] = y
    elif m == 1:
        out_a[:, 128:192] = y[:, 0:64]
        out_b[:, 0:64] = y[:, 64:128]
    else:
        out_b[:, 64:192] = y


def _prep_items(v_feat, t_feat, coll, wv, bv, wt, bt, wc, bc):
    R = 2000

    def body(v_ref, t_ref, c_ref, wv_ref, bv_ref, wt_ref, bt_ref,
             wc_ref, bc_ref, out_a, out_b):
        for m, (f, w, b) in enumerate((
                (v_ref, wv_ref, bv_ref), (t_ref, wt_ref, bt_ref),
                (c_ref, wc_ref, bc_ref))):
            y = jnp.dot(f[...], w[...], preferred_element_type=jnp.float32)
            y = y + b[...]
            _split_write(out_a, out_b, m, _normalize(y))

    return pl.pallas_call(
        body,
        grid=(NI // R,),
        in_specs=[_rows((R, 256)), _rows((R, 128)), _rows((R, 64)),
                  _full((256, 128)), _full((1, 128)),
                  _full((128, 128)), _full((1, 128)),
                  _full((64, 128)), _full((1, 128))],
        out_specs=(_rows((R, 192)), _rows((R, 192))),
        out_shape=(jax.ShapeDtypeStruct((NI, 192), jnp.float32),
                   jax.ShapeDtypeStruct((NI, 192), jnp.float32)),
    )(v_feat, t_feat, coll, wv, bv, wt, bt, wc, bc)


def _prep_users(pv, pt, pc):
    R = 2000

    def body(pv_ref, pt_ref, pc_ref, out_a, out_b):
        for m, p in enumerate((pv_ref, pt_ref, pc_ref)):
            _split_write(out_a, out_b, m, _normalize(p[...]))

    return pl.pallas_call(
        body,
        grid=(NU // R,),
        in_specs=[_rows((R, 128))] * 3,
        out_specs=(_rows((R, 192)), _rows((R, 192))),
        out_shape=(jax.ShapeDtypeStruct((NU, 192), jnp.float32),
                   jax.ShapeDtypeStruct((NU, 192), jnp.float32)),
    )(pv, pt, pc)


def _layer(a, W, G, Gb):
    rows, win = a.shape
    R = 2000

    def body(a_ref, w_ref, g_ref, gb_ref, out_ref):
        h = _lrelu(jnp.dot(a_ref[...], w_ref[...],
                           preferred_element_type=jnp.float32))
        x = _lrelu(jnp.dot(h, g_ref[...],
                           preferred_element_type=jnp.float32) + gb_ref[...])
        out_ref[...] = x

    return pl.pallas_call(
        body,
        grid=(rows // R,),
        in_specs=[_rows((R, win)), _full((win, 192)), _full((192, 192)),
                  _full((1, 192))],
        out_specs=_rows((R, 192)),
        out_shape=jax.ShapeDtypeStruct((rows, 192), jnp.float32),
    )(a, W, G, Gb)


def _layer2in(aa, ab, Wa, Wb, G, Gb):
    rows = aa.shape[0]
    R = 2000

    def body(aa_ref, ab_ref, wa_ref, wb_ref, g_ref, gb_ref, out_ref):
        h = jnp.dot(aa_ref[...], wa_ref[...],
                    preferred_element_type=jnp.float32)
        h = h + jnp.dot(ab_ref[...], wb_ref[...],
                        preferred_element_type=jnp.float32)
        h = _lrelu(h)
        x = _lrelu(jnp.dot(h, g_ref[...],
                           preferred_element_type=jnp.float32) + gb_ref[...])
        out_ref[...] = x

    return pl.pallas_call(
        body,
        grid=(rows // R,),
        in_specs=[_rows((R, 192)), _rows((R, 192)), _full((192, 192)),
                  _full((192, 192)), _full((192, 192)), _full((1, 192))],
        out_specs=_rows((R, 192)),
        out_shape=jax.ShapeDtypeStruct((rows, 192), jnp.float32),
    )(aa, ab, Wa, Wb, G, Gb)


def _heads(a, x, W4, G4, L4, W5, G5, L5, G4b, L4b, G5b, L5b):
    rows = a.shape[0]
    R = 2000
    eps = 1e-8

    def body(a_ref, x_ref, w4, g4, l4, w5, g5, l5, g4b, l4b, g5b, l5b,
             pm_ref, plv_ref):
        av = a_ref[...]
        xv = x_ref[...]
        dot = lambda p, q: jnp.dot(p, q, preferred_element_type=jnp.float32)
        mu = dot(_lrelu(dot(av, w4[...])), g4[...]) + g4b[...] \
            + _lrelu(dot(xv, l4[...]) + l4b[...])
        lv = dot(_lrelu(dot(av, w5[...])), g5[...]) + g5b[...] \
            + _lrelu(dot(xv, l5[...]) + l5b[...])

        def poe2(m1, l1, m2, l2):
            t1 = 1.0 / (jnp.exp(l1) + eps)
            t2 = 1.0 / (jnp.exp(l2) + eps)
            pm = (m1 * t1 + m2 * t2) / (t1 + t2)
            return pm, jnp.log(1.0 / (t1 + t2))

        pm, plv = poe2(mu[:, 0:64], lv[:, 0:64], mu[:, 64:128], lv[:, 64:128])
        pm, plv = poe2(pm, plv, mu[:, 128:192], lv[:, 128:192])
        pm_ref[...] = pm
        plv_ref[...] = jnp.minimum(plv, MAX_LOGVAR)

    return pl.pallas_call(
        body,
        grid=(rows // R,),
        in_specs=[_rows((R, 192)), _rows((R, 192)),
                  _full((192, 192)), _full((192, 192)), _full((192, 192)),
                  _full((192, 192)), _full((192, 192)), _full((192, 192)),
                  _full((1, 192)), _full((1, 192)), _full((1, 192)),
                  _full((1, 192))],
        out_specs=(_rows((R, 64)), _rows((R, 64))),
        out_shape=(jax.ShapeDtypeStruct((rows, 64), jnp.float32),
                   jax.ShapeDtypeStruct((rows, 64), jnp.float32)),
    )(a, x, W4, G4, L4, W5, G5, L5, G4b, L4b, G5b, L5b)


def _bd(ws):
    r = sum(w.shape[0] for w in ws)
    c = sum(w.shape[1] for w in ws)
    out = jnp.zeros((r, c), jnp.float32)
    ro = co = 0
    for w in ws:
        out = out.at[ro:ro + w.shape[0], co:co + w.shape[1]].set(w)
        ro += w.shape[0]
        co += w.shape[1]
    return out


def _cat_b(ps, name):
    return jnp.concatenate([p[name] for p in ps])[None, :]


def kernel(v_feat, t_feat, collaborative, edge_index, params):
    src = edge_index[:, 0]
    dloc = edge_index[:, 1] - NU
    pad = jnp.full((EPAD - E,), SENT, jnp.int32)
    srcp = jnp.concatenate([src, pad])
    dlocp = jnp.concatenate([dloc, pad])

    ps = [params['v'], params['t'], params['c']]

    XuA, XuB = _prep_users(*[p['preference'] for p in ps])
    XiA, XiB = _prep_items(v_feat, t_feat, collaborative,
                           ps[0]['mlp_w'], ps[0]['mlp_b'][None, :],
                           ps[1]['mlp_w'], ps[1]['mlp_b'][None, :],
                           ps[2]['mlp_w'], ps[2]['mlp_b'][None, :])

    W1 = _bd([p['conv1_w'] for p in ps])
    G1 = _bd([p['g1_w'] for p in ps]); G1b = _cat_b(ps, 'g1_b')
    W2 = _bd([p['conv2_w'] for p in ps])
    G2 = _bd([p['g2_w'] for p in ps]); G2b = _cat_b(ps, 'g2_b')
    W4 = _bd([p['conv4_w'] for p in ps])
    G4 = _bd([p['g4_w'] for p in ps]); G4b = _cat_b(ps, 'g4_b')
    L4 = _bd([p['lin4_w'] for p in ps]); L4b = _cat_b(ps, 'lin4_b')
    W5 = _bd([p['conv5_w'] for p in ps])
    G5 = _bd([p['g5_w'] for p in ps]); G5b = _cat_b(ps, 'g5_b')
    L5 = _bd([p['lin5_w'] for p in ps]); L5b = _cat_b(ps, 'lin5_b')

    aUA, aIA = _agg192(XuA, XiA, srcp, dlocp)
    aUB, aIB = _agg192(XuB, XiB, srcp, dlocp)
    W1a, W1b = W1[:192], W1[192:]
    Xu = _layer2in(aUA, aUB, W1a, W1b, G1, G1b)
    Xi = _layer2in(aIA, aIB, W1a, W1b, G1, G1b)

    aU, aI = _agg192(Xu, Xi, srcp, dlocp)
    Xu2 = _layer(aU, W2, G2, G2b)
    Xi2 = _layer(aI, W2, G2, G2b)

    aU, aI = _agg192(Xu2, Xi2, srcp, dlocp)
    pmU, plvU = _heads(aU, Xu2, W4, G4, L4, W5, G5, L5, G4b, L4b, G5b, L5b)
    pmI, plvI = _heads(aI, Xi2, W4, G4, L4, W5, G5, L5, G4b, L4b, G5b, L5b)

    pm = jnp.concatenate([pmU, pmI], axis=0)
    plv = jnp.concatenate([plvU, plvI], axis=0)
    return pm, pm, plv


# speedup vs baseline: 1.6997x; 1.6997x over previous
import functools

import jax
import jax.numpy as jnp
from jax import lax
from jax.experimental import pallas as pl
from jax.experimental.pallas import tpu as pltpu
from jax.experimental.pallas import tpu_sc as plsc

NU = 10000
NI = 40000
E = 500000
EPAD = 524288
PER_TILE = EPAD // 16
NSB = 16
S = PER_TILE // NSB
CH = 5000
SPM_ROWS = CH + 8
TPR = 312
SENT = 1 << 20
MAX_LOGVAR = 10.0


def _lrelu(x):
    return jnp.where(x >= 0, x, 0.01 * x)


B = 96
NBK = (S + B - 1) // B


def _make_agg(W):
    mesh = plsc.VectorSubcoreMesh(core_axis_name="c", subcore_axis_name="s")

    def body(xu_hbm, xi_hbm, src_hbm, dloc_hbm, zeros_hbm,
             aggu_hbm, aggi_hbm,
             src_v, dloc_v, gidx, sidx,
             rb0, rb1, rb2, si0, si1, si2, spm,
             gs0, gs1, gs2, ss0, ss1, ss2):
        core = lax.axis_index("c")
        sub = lax.axis_index("s")
        ebase = sub * PER_TILE
        start = pl.multiple_of(sub * TPR, 8)
        rbufs = (rb0, rb1, rb2)
        sbufs = (si0, si1, si2)
        gsems = (gs0, gs1, gs2)
        ssems = (ss0, ss1, ss2)

        def do_chunk(lo, scat_ref, gath_ref, tab_hbm, out_hbm):
            pltpu.sync_copy(zeros_hbm, spm.at[pl.ds(start, 320)])
            plsc.subcore_barrier()

            def do_sb(sb, _):
                eoff = ebase + sb * S
                pltpu.sync_copy(src_hbm.at[pl.ds(eoff, S)], src_v)
                pltpu.sync_copy(dloc_hbm.at[pl.ds(eoff, S)], dloc_v)

                def scan_v(v, n):
                    sc = scat_ref[pl.ds(v * 16, 16)] - lo
                    g = gath_ref[pl.ds(v * 16, 16)]
                    m = (sc >= 0) & (sc < CH)
                    plsc.store_compressed(sidx.at[pl.ds(n, 16)], sc, mask=m)
                    plsc.store_compressed(gidx.at[pl.ds(n, 16)], g, mask=m)
                    return n + jnp.sum(m.astype(jnp.int32))
                n = lax.fori_loop(0, S // 16, scan_v, 0)

                for j in range(B // 16):
                    sidx[pl.ds(n + j * 16, 16)] = jnp.full((16,), CH,
                                                           jnp.int32)
                    gidx[pl.ds(n + j * 16, 16)] = jnp.zeros((16,), jnp.int32)

                def fire_gather(k):
                    s = k % 3
                    pltpu.async_copy(tab_hbm.at[gidx.at[pl.ds(k * B, B)]],
                                     rbufs[s], gsems[s])

                def wait_gather(k):
                    s = k % 3
                    pltpu.make_async_copy(
                        tab_hbm.at[gidx.at[pl.ds(k * B, B)]],
                        rbufs[s], gsems[s]).wait()

                def fire_scatter(k):
                    s = k % 3
                    for j in range(B // 16):
                        sbufs[s][pl.ds(j * 16, 16)] = \
                            sidx[pl.ds(k * B + j * 16, 16)]
                    pltpu.async_copy(rbufs[s], spm.at[sbufs[s]], ssems[s],
                                     add=True)

                def wait_scatter(k):
                    s = k % 3
                    pltpu.make_async_copy(rbufs[s], spm.at[sbufs[s]],
                                          ssems[s]).wait()

                @pl.when(0 < n)
                def _():
                    fire_gather(0)
                for k in range(NBK):
                    if k >= 2:
                        @pl.when((k - 2) * B < n)
                        def _(k=k):
                            wait_scatter(k - 2)
                    if k + 1 < NBK:
                        @pl.when((k + 1) * B < n)
                        def _(k=k):
                            fire_gather(k + 1)

                    @pl.when(k * B < n)
                    def _(k=k):
                        wait_gather(k)
                        fire_scatter(k)
                for k in (NBK - 2, NBK - 1):
                    @pl.when(k * B < n)
                    def _(k=k):
                        wait_scatter(k)
                return 0
            lax.fori_loop(0, NSB, do_sb, 0)
            plsc.subcore_barrier()

            pltpu.sync_copy(spm.at[pl.ds(start, 320)],
                            out_hbm.at[pl.ds(lo + start, 320)])
            plsc.subcore_barrier()

        do_chunk(core * CH, src_v, dloc_v, xi_hbm, aggu_hbm)
        for i in range(4):
            do_chunk((core + 2 * i) * CH, dloc_v, src_v, xu_hbm, aggi_hbm)

    kern = pl.kernel(
        body,
        out_type=(jax.ShapeDtypeStruct((NU, W), jnp.float32),
                  jax.ShapeDtypeStruct((NI, W), jnp.float32)),
        mesh=mesh,
        scratch_types=[
            pltpu.VMEM((S,), jnp.int32),
            pltpu.VMEM((S,), jnp.int32),
            pltpu.VMEM((S + B, ), jnp.int32),
            pltpu.VMEM((S + B, ), jnp.int32),
            pltpu.VMEM((B, W), jnp.float32),
            pltpu.VMEM((B, W), jnp.float32),
            pltpu.VMEM((B, W), jnp.float32),
            pltpu.VMEM((B,), jnp.int32),
            pltpu.VMEM((B,), jnp.int32),
            pltpu.VMEM((B,), jnp.int32),
            pltpu.VMEM_SHARED((SPM_ROWS, W), jnp.float32),
            pltpu.SemaphoreType.DMA,
            pltpu.SemaphoreType.DMA,
            pltpu.SemaphoreType.DMA,
            pltpu.SemaphoreType.DMA,
            pltpu.SemaphoreType.DMA,
            pltpu.SemaphoreType.DMA,
        ],
        compiler_params=pltpu.CompilerParams(needs_layout_passes=False,
                                             use_tc_tiling_on_sc=False),
    )
    return kern


_agg192 = _make_agg(192)


def _full(shape):
    return pl.BlockSpec(shape, lambda i: tuple(0 for _ in shape))


def _rows(bshape):
    return pl.BlockSpec(bshape, lambda i: (i,) + tuple(0 for _ in bshape[1:]))


def _normalize(y):
    nrm = jnp.maximum(jnp.sqrt(jnp.sum(y * y, axis=1, keepdims=True)), 1e-12)
    return y / nrm


def _split_write(out_a, out_b, m, y):
    if m == 0:
        out_a[:, 0:128] = y
    elif m == 1:
        out_a[:, 128:192] = y[:, 0:64]
        out_b[:, 0:64] = y[:, 64:128]
    else:
        out_b[:, 64:192] = y


def _prep_items(v_feat, t_feat, coll, wv, bv, wt, bt, wc, bc):
    R = 2000

    def body(v_ref, t_ref, c_ref, wv_ref, bv_ref, wt_ref, bt_ref,
             wc_ref, bc_ref, out_a, out_b):
        for m, (f, w, b) in enumerate((
                (v_ref, wv_ref, bv_ref), (t_ref, wt_ref, bt_ref),
                (c_ref, wc_ref, bc_ref))):
            y = jnp.dot(f[...], w[...], preferred_element_type=jnp.float32)
            y = y + b[...]
            _split_write(out_a, out_b, m, _normalize(y))

    return pl.pallas_call(
        body,
        grid=(NI // R,),
        in_specs=[_rows((R, 256)), _rows((R, 128)), _rows((R, 64)),
                  _full((256, 128)), _full((1, 128)),
                  _full((128, 128)), _full((1, 128)),
                  _full((64, 128)), _full((1, 128))],
        out_specs=(_rows((R, 192)), _rows((R, 192))),
        out_shape=(jax.ShapeDtypeStruct((NI, 192), jnp.float32),
                   jax.ShapeDtypeStruct((NI, 192), jnp.float32)),
    )(v_feat, t_feat, coll, wv, bv, wt, bt, wc, bc)


def _prep_users(pv, pt, pc):
    R = 2000

    def body(pv_ref, pt_ref, pc_ref, out_a, out_b):
        for m, p in enumerate((pv_ref, pt_ref, pc_ref)):
            _split_write(out_a, out_b, m, _normalize(p[...]))

    return pl.pallas_call(
        body,
        grid=(NU // R,),
        in_specs=[_rows((R, 128))] * 3,
        out_specs=(_rows((R, 192)), _rows((R, 192))),
        out_shape=(jax.ShapeDtypeStruct((NU, 192), jnp.float32),
                   jax.ShapeDtypeStruct((NU, 192), jnp.float32)),
    )(pv, pt, pc)


def _layer(a, W, G, Gb):
    rows, win = a.shape
    R = 2000

    def body(a_ref, w_ref, g_ref, gb_ref, out_ref):
        h = _lrelu(jnp.dot(a_ref[...], w_ref[...],
                           preferred_element_type=jnp.float32))
        x = _lrelu(jnp.dot(h, g_ref[...],
                           preferred_element_type=jnp.float32) + gb_ref[...])
        out_ref[...] = x

    return pl.pallas_call(
        body,
        grid=(rows // R,),
        in_specs=[_rows((R, win)), _full((win, 192)), _full((192, 192)),
                  _full((1, 192))],
        out_specs=_rows((R, 192)),
        out_shape=jax.ShapeDtypeStruct((rows, 192), jnp.float32),
    )(a, W, G, Gb)


def _layer2in(aa, ab, Wa, Wb, G, Gb):
    rows = aa.shape[0]
    R = 2000

    def body(aa_ref, ab_ref, wa_ref, wb_ref, g_ref, gb_ref, out_ref):
        h = jnp.dot(aa_ref[...], wa_ref[...],
                    preferred_element_type=jnp.float32)
        h = h + jnp.dot(ab_ref[...], wb_ref[...],
                        preferred_element_type=jnp.float32)
        h = _lrelu(h)
        x = _lrelu(jnp.dot(h, g_ref[...],
                           preferred_element_type=jnp.float32) + gb_ref[...])
        out_ref[...] = x

    return pl.pallas_call(
        body,
        grid=(rows // R,),
        in_specs=[_rows((R, 192)), _rows((R, 192)), _full((192, 192)),
                  _full((192, 192)), _full((192, 192)), _full((1, 192))],
        out_specs=_rows((R, 192)),
        out_shape=jax.ShapeDtypeStruct((rows, 192), jnp.float32),
    )(aa, ab, Wa, Wb, G, Gb)


def _heads(a, x, W4, G4, L4, W5, G5, L5, G4b, L4b, G5b, L5b):
    rows = a.shape[0]
    R = 2000
    eps = 1e-8

    def body(a_ref, x_ref, w4, g4, l4, w5, g5, l5, g4b, l4b, g5b, l5b,
             pm_ref, plv_ref):
        av = a_ref[...]
        xv = x_ref[...]
        dot = lambda p, q: jnp.dot(p, q, preferred_element_type=jnp.float32)
        mu = dot(_lrelu(dot(av, w4[...])), g4[...]) + g4b[...] \
            + _lrelu(dot(xv, l4[...]) + l4b[...])
        lv = dot(_lrelu(dot(av, w5[...])), g5[...]) + g5b[...] \
            + _lrelu(dot(xv, l5[...]) + l5b[...])

        def poe2(m1, l1, m2, l2):
            t1 = 1.0 / (jnp.exp(l1) + eps)
            t2 = 1.0 / (jnp.exp(l2) + eps)
            pm = (m1 * t1 + m2 * t2) / (t1 + t2)
            return pm, jnp.log(1.0 / (t1 + t2))

        pm, plv = poe2(mu[:, 0:64], lv[:, 0:64], mu[:, 64:128], lv[:, 64:128])
        pm, plv = poe2(pm, plv, mu[:, 128:192], lv[:, 128:192])
        pm_ref[...] = pm
        plv_ref[...] = jnp.minimum(plv, MAX_LOGVAR)

    return pl.pallas_call(
        body,
        grid=(rows // R,),
        in_specs=[_rows((R, 192)), _rows((R, 192)),
                  _full((192, 192)), _full((192, 192)), _full((192, 192)),
                  _full((192, 192)), _full((192, 192)), _full((192, 192)),
                  _full((1, 192)), _full((1, 192)), _full((1, 192)),
                  _full((1, 192))],
        out_specs=(_rows((R, 64)), _rows((R, 64))),
        out_shape=(jax.ShapeDtypeStruct((rows, 64), jnp.float32),
                   jax.ShapeDtypeStruct((rows, 64), jnp.float32)),
    )(a, x, W4, G4, L4, W5, G5, L5, G4b, L4b, G5b, L5b)


def _bd(ws):
    r = sum(w.shape[0] for w in ws)
    c = sum(w.shape[1] for w in ws)
    out = jnp.zeros((r, c), jnp.float32)
    ro = co = 0
    for w in ws:
        out = out.at[ro:ro + w.shape[0], co:co + w.shape[1]].set(w)
        ro += w.shape[0]
        co += w.shape[1]
    return out


def _cat_b(ps, name):
    return jnp.concatenate([p[name] for p in ps])[None, :]


def kernel(v_feat, t_feat, collaborative, edge_index, params):
    src = edge_index[:, 0]
    dloc = edge_index[:, 1] - NU
    pad = jnp.full((EPAD - E,), SENT, jnp.int32)
    srcp = jnp.concatenate([src, pad])
    dlocp = jnp.concatenate([dloc, pad])

    ps = [params['v'], params['t'], params['c']]

    XuA, XuB = _prep_users(*[p['preference'] for p in ps])
    XiA, XiB = _prep_items(v_feat, t_feat, collaborative,
                           ps[0]['mlp_w'], ps[0]['mlp_b'][None, :],
                           ps[1]['mlp_w'], ps[1]['mlp_b'][None, :],
                           ps[2]['mlp_w'], ps[2]['mlp_b'][None, :])

    W1 = _bd([p['conv1_w'] for p in ps])
    G1 = _bd([p['g1_w'] for p in ps]); G1b = _cat_b(ps, 'g1_b')
    W2 = _bd([p['conv2_w'] for p in ps])
    G2 = _bd([p['g2_w'] for p in ps]); G2b = _cat_b(ps, 'g2_b')
    W4 = _bd([p['conv4_w'] for p in ps])
    G4 = _bd([p['g4_w'] for p in ps]); G4b = _cat_b(ps, 'g4_b')
    L4 = _bd([p['lin4_w'] for p in ps]); L4b = _cat_b(ps, 'lin4_b')
    W5 = _bd([p['conv5_w'] for p in ps])
    G5 = _bd([p['g5_w'] for p in ps]); G5b = _cat_b(ps, 'g5_b')
    L5 = _bd([p['lin5_w'] for p in ps]); L5b = _cat_b(ps, 'lin5_b')

    Z = jnp.zeros((320, 192), jnp.float32)
    aUA, aIA = _agg192(XuA, XiA, srcp, dlocp, Z)
    aUB, aIB = _agg192(XuB, XiB, srcp, dlocp, Z)
    W1a, W1b = W1[:192], W1[192:]
    Xu = _layer2in(aUA, aUB, W1a, W1b, G1, G1b)
    Xi = _layer2in(aIA, aIB, W1a, W1b, G1, G1b)

    aU, aI = _agg192(Xu, Xi, srcp, dlocp, Z)
    Xu2 = _layer(aU, W2, G2, G2b)
    Xi2 = _layer(aI, W2, G2, G2b)

    aU, aI = _agg192(Xu2, Xi2, srcp, dlocp, Z)
    pmU, plvU = _heads(aU, Xu2, W4, G4, L4, W5, G5, L5, G4b, L4b, G5b, L5b)
    pmI, plvI = _heads(aI, Xi2, W4, G4, L4, W5, G5, L5, G4b, L4b, G5b, L5b)

    pm = jnp.concatenate([pmU, pmI], axis=0)
    plv = jnp.concatenate([plvU, plvI], axis=0)
    return pm, pm, plv


# speedup vs baseline: 3.8496x; 2.2648x over previous
import jax
import jax.numpy as jnp
from jax import lax
from jax.experimental import pallas as pl
from jax.experimental.pallas import tpu as pltpu
from jax.experimental.pallas import tpu_sc as plsc

NU = 10000
NI = 40000
E = 500000
EPAD = 524288
PER_TILE = EPAD // 16
NSB = 16
S = PER_TILE // NSB
RING = 8
SENT = 1 << 20
MAX_LOGVAR = 10.0


def _lrelu(x):
    return jnp.where(x >= 0, x, 0.01 * x)


def _make_agg(W, CH, B, NUP, NIP):
    stride = (CH // 16 // 8) * 8
    cover = CH - 15 * stride
    assert cover % 8 == 0 and cover >= stride and 15 * stride + cover == CH
    NBK = S // B
    assert NBK % RING == 0
    NUC, NIC = NUP // CH, NIP // CH
    assert NUC * CH == NUP and NIC * CH == NIP and NUC % 2 == 0 and NIC % 2 == 0

    mesh = plsc.VectorSubcoreMesh(core_axis_name="c", subcore_axis_name="s")

    def body(xu_hbm, xi_hbm, src_hbm, dloc_hbm, zeros_hbm,
             aggu_hbm, aggi_hbm,
             src_v, dloc_v, gidx, sidx,
             rbufs, sbufs, spm, gsems, ssems):
        core = lax.axis_index("c")
        sub = lax.axis_index("s")
        ebase = sub * PER_TILE
        start = pl.multiple_of(sub * stride, 8)

        def do_chunk(lo, scat_ref, gath_ref, tab_hbm, out_hbm):
            pltpu.sync_copy(zeros_hbm, spm.at[pl.ds(start, cover)])
            plsc.subcore_barrier()

            def do_sb(sb, _):
                eoff = ebase + sb * S
                pltpu.sync_copy(src_hbm.at[pl.ds(eoff, S)], src_v)
                pltpu.sync_copy(dloc_hbm.at[pl.ds(eoff, S)], dloc_v)

                def scan_v(v, n):
                    sc = scat_ref[pl.ds(v * 16, 16)] - lo
                    g = gath_ref[pl.ds(v * 16, 16)]
                    m = (sc >= 0) & (sc < CH)
                    plsc.store_compressed(sidx.at[pl.ds(n, 16)], sc, mask=m)
                    plsc.store_compressed(gidx.at[pl.ds(n, 16)], g, mask=m)
                    return n + jnp.sum(m.astype(jnp.int32))
                n = lax.fori_loop(0, S // 16, scan_v, 0)

                for j in range(B // 16):
                    sidx[pl.ds(n + j * 16, 16)] = jnp.full((16,), CH,
                                                           jnp.int32)
                    gidx[pl.ds(n + j * 16, 16)] = jnp.zeros((16,), jnp.int32)

                def fire_gather(k, s):
                    pltpu.async_copy(tab_hbm.at[gidx.at[pl.ds(k * B, B)]],
                                     rbufs[s], gsems[s])

                def wait_gather(k, s):
                    pltpu.make_async_copy(
                        tab_hbm.at[gidx.at[pl.ds(k * B, B)]],
                        rbufs[s], gsems[s]).wait()

                def fire_scatter(k, s):
                    for j in range(B // 16):
                        sbufs[s][pl.ds(j * 16, 16)] = \
                            sidx[pl.ds(k * B + j * 16, 16)]
                    pltpu.async_copy(rbufs[s], spm.at[sbufs[s]], ssems[s],
                                     add=True)

                def wait_scatter(s):
                    pltpu.make_async_copy(rbufs[s], spm.at[sbufs[s]],
                                          ssems[s]).wait()

                LEAD = RING - 2
                for k in range(LEAD):
                    @pl.when(k * B < n)
                    def _(k=k):
                        fire_gather(k, k % RING)

                def group(g, _):
                    for j in range(RING):
                        k = g * RING + j

                        @pl.when((k >= 2) & ((k - 2) * B < n))
                        def _(j=j):
                            wait_scatter((j - 2) % RING)

                        @pl.when((k + LEAD < NBK) & ((k + LEAD) * B < n))
                        def _(k=k, j=j):
                            fire_gather(k + LEAD, (j + LEAD) % RING)

                        @pl.when(k * B < n)
                        def _(k=k, j=j):
                            wait_gather(k, j % RING)
                            fire_scatter(k, j % RING)
                    return 0
                lax.fori_loop(0, NBK // RING, group, 0)
                for k in (NBK - 2, NBK - 1):
                    @pl.when(k * B < n)
                    def _(k=k):
                        wait_scatter(k % RING)
                return 0
            lax.fori_loop(0, NSB, do_sb, 0)
            plsc.subcore_barrier()

            pltpu.sync_copy(spm.at[pl.ds(start, cover)],
                            out_hbm.at[pl.ds(lo + start, cover)])
            plsc.subcore_barrier()

        for i in range(NUC // 2):
            do_chunk((core + 2 * i) * CH, src_v, dloc_v, xi_hbm, aggu_hbm)
        for i in range(NIC // 2):
            do_chunk((core + 2 * i) * CH, dloc_v, src_v, xu_hbm, aggi_hbm)

    kern = pl.kernel(
        body,
        out_type=(jax.ShapeDtypeStruct((NUP, W), jnp.float32),
                  jax.ShapeDtypeStruct((NIP, W), jnp.float32)),
        mesh=mesh,
        scratch_types=[
            pltpu.VMEM((S,), jnp.int32),
            pltpu.VMEM((S,), jnp.int32),
            pltpu.VMEM((S + B, ), jnp.int32),
            pltpu.VMEM((S + B, ), jnp.int32),
            tuple(pltpu.VMEM((B, W), jnp.float32) for _ in range(RING)),
            tuple(pltpu.VMEM((B,), jnp.int32) for _ in range(RING)),
            pltpu.VMEM_SHARED((CH + 8, W), jnp.float32),
            tuple(pltpu.SemaphoreType.DMA for _ in range(RING)),
            tuple(pltpu.SemaphoreType.DMA for _ in range(RING)),
        ],
        compiler_params=pltpu.CompilerParams(needs_layout_passes=False,
                                             use_tc_tiling_on_sc=False),
    )
    return kern, cover


NUP1, NIP1 = 10016, 40064
_agg384, _cov384 = _make_agg(384, 2504, 16, NUP1, NIP1)
_agg192, _cov192 = _make_agg(192, 5000, 32, NU, NI)


def _full(shape):
    return pl.BlockSpec(shape, lambda i: tuple(0 for _ in shape))


def _rows(bshape):
    return pl.BlockSpec(bshape, lambda i: (i,) + tuple(0 for _ in bshape[1:]))


def _normalize(y):
    nrm = jnp.maximum(jnp.sqrt(jnp.sum(y * y, axis=1, keepdims=True)), 1e-12)
    return y / nrm


def _prep_items(v_feat, t_feat, coll, wv, bv, wt, bt, wc, bc):
    R = 2000

    def body(v_ref, t_ref, c_ref, wv_ref, bv_ref, wt_ref, bt_ref,
             wc_ref, bc_ref, out_ref):
        for m, (f, w, b) in enumerate((
                (v_ref, wv_ref, bv_ref), (t_ref, wt_ref, bt_ref),
                (c_ref, wc_ref, bc_ref))):
            y = jnp.dot(f[...], w[...], preferred_element_type=jnp.float32)
            y = y + b[...]
            out_ref[:, m * 128:(m + 1) * 128] = _normalize(y)

    return pl.pallas_call(
        body,
        grid=(NI // R,),
        in_specs=[_rows((R, 256)), _rows((R, 128)), _rows((R, 64)),
                  _full((256, 128)), _full((1, 128)),
                  _full((128, 128)), _full((1, 128)),
                  _full((64, 128)), _full((1, 128))],
        out_specs=_rows((R, 384)),
        out_shape=jax.ShapeDtypeStruct((NI, 384), jnp.float32),
    )(v_feat, t_feat, coll, wv, bv, wt, bt, wc, bc)


def _prep_users(pv, pt, pc):
    R = 2000

    def body(pv_ref, pt_ref, pc_ref, out_ref):
        for m, p in enumerate((pv_ref, pt_ref, pc_ref)):
            out_ref[:, m * 128:(m + 1) * 128] = _normalize(p[...])

    return pl.pallas_call(
        body,
        grid=(NU // R,),
        in_specs=[_rows((R, 128))] * 3,
        out_specs=_rows((R, 384)),
        out_shape=jax.ShapeDtypeStruct((NU, 384), jnp.float32),
    )(pv, pt, pc)


def _layer(a, W, G, Gb):
    rows, win = a.shape
    R = 2000

    def body(a_ref, w_ref, g_ref, gb_ref, out_ref):
        h = _lrelu(jnp.dot(a_ref[...], w_ref[...],
                           preferred_element_type=jnp.float32))
        x = _lrelu(jnp.dot(h, g_ref[...],
                           preferred_element_type=jnp.float32) + gb_ref[...])
        out_ref[...] = x

    return pl.pallas_call(
        body,
        grid=(rows // R,),
        in_specs=[_rows((R, win)), _full((win, 192)), _full((192, 192)),
                  _full((1, 192))],
        out_specs=_rows((R, 192)),
        out_shape=jax.ShapeDtypeStruct((rows, 192), jnp.float32),
    )(a, W, G, Gb)


def _heads(a, x, W4, G4, L4, W5, G5, L5, G4b, L4b, G5b, L5b):
    rows = a.shape[0]
    R = 2000
    eps = 1e-8

    def body(a_ref, x_ref, w4, g4, l4, w5, g5, l5, g4b, l4b, g5b, l5b,
             pm_ref, plv_ref):
        av = a_ref[...]
        xv = x_ref[...]
        dot = lambda p, q: jnp.dot(p, q, preferred_element_type=jnp.float32)
        mu = dot(_lrelu(dot(av, w4[...])), g4[...]) + g4b[...] \
            + _lrelu(dot(xv, l4[...]) + l4b[...])
        lv = dot(_lrelu(dot(av, w5[...])), g5[...]) + g5b[...] \
            + _lrelu(dot(xv, l5[...]) + l5b[...])

        def poe2(m1, l1, m2, l2):
            t1 = 1.0 / (jnp.exp(l1) + eps)
            t2 = 1.0 / (jnp.exp(l2) + eps)
            pm = (m1 * t1 + m2 * t2) / (t1 + t2)
            return pm, jnp.log(1.0 / (t1 + t2))

        pm, plv = poe2(mu[:, 0:64], lv[:, 0:64], mu[:, 64:128], lv[:, 64:128])
        pm, plv = poe2(pm, plv, mu[:, 128:192], lv[:, 128:192])
        pm_ref[...] = pm
        plv_ref[...] = jnp.minimum(plv, MAX_LOGVAR)

    return pl.pallas_call(
        body,
        grid=(rows // R,),
        in_specs=[_rows((R, 192)), _rows((R, 192)),
                  _full((192, 192)), _full((192, 192)), _full((192, 192)),
                  _full((192, 192)), _full((192, 192)), _full((192, 192)),
                  _full((1, 192)), _full((1, 192)), _full((1, 192)),
                  _full((1, 192))],
        out_specs=(_rows((R, 64)), _rows((R, 64))),
        out_shape=(jax.ShapeDtypeStruct((rows, 64), jnp.float32),
                   jax.ShapeDtypeStruct((rows, 64), jnp.float32)),
    )(a, x, W4, G4, L4, W5, G5, L5, G4b, L4b, G5b, L5b)


def _bd(ws):
    r = sum(w.shape[0] for w in ws)
    c = sum(w.shape[1] for w in ws)
    out = jnp.zeros((r, c), jnp.float32)
    ro = co = 0
    for w in ws:
        out = out.at[ro:ro + w.shape[0], co:co + w.shape[1]].set(w)
        ro += w.shape[0]
        co += w.shape[1]
    return out


def _cat_b(ps, name):
    return jnp.concatenate([p[name] for p in ps])[None, :]


def kernel(v_feat, t_feat, collaborative, edge_index, params):
    src = edge_index[:, 0]
    dloc = edge_index[:, 1] - NU
    pad = jnp.full((EPAD - E,), SENT, jnp.int32)
    srcp = jnp.concatenate([src, pad])
    dlocp = jnp.concatenate([dloc, pad])

    ps = [params['v'], params['t'], params['c']]

    Xu = _prep_users(*[p['preference'] for p in ps])
    Xi = _prep_items(v_feat, t_feat, collaborative,
                     ps[0]['mlp_w'], ps[0]['mlp_b'][None, :],
                     ps[1]['mlp_w'], ps[1]['mlp_b'][None, :],
                     ps[2]['mlp_w'], ps[2]['mlp_b'][None, :])

    W1 = _bd([p['conv1_w'] for p in ps])
    G1 = _bd([p['g1_w'] for p in ps]); G1b = _cat_b(ps, 'g1_b')
    W2 = _bd([p['conv2_w'] for p in ps])
    G2 = _bd([p['g2_w'] for p in ps]); G2b = _cat_b(ps, 'g2_b')
    W4 = _bd([p['conv4_w'] for p in ps])
    G4 = _bd([p['g4_w'] for p in ps]); G4b = _cat_b(ps, 'g4_b')
    L4 = _bd([p['lin4_w'] for p in ps]); L4b = _cat_b(ps, 'lin4_b')
    W5 = _bd([p['conv5_w'] for p in ps])
    G5 = _bd([p['g5_w'] for p in ps]); G5b = _cat_b(ps, 'g5_b')
    L5 = _bd([p['lin5_w'] for p in ps]); L5b = _cat_b(ps, 'lin5_b')

    Z384 = jnp.zeros((_cov384, 384), jnp.float32)
    Z192 = jnp.zeros((_cov192, 192), jnp.float32)

    aU, aI = _agg384(Xu, Xi, srcp, dlocp, Z384)
    aU, aI = aU[:NU], aI[:NI]
    Xu = _layer(aU, W1, G1, G1b)
    Xi = _layer(aI, W1, G1, G1b)

    aU, aI = _agg192(Xu, Xi, srcp, dlocp, Z192)
    Xu2 = _layer(aU, W2, G2, G2b)
    Xi2 = _layer(aI, W2, G2, G2b)

    aU, aI = _agg192(Xu2, Xi2, srcp, dlocp, Z192)
    pmU, plvU = _heads(aU, Xu2, W4, G4, L4, W5, G5, L5, G4b, L4b, G5b, L5b)
    pmI, plvI = _heads(aI, Xi2, W4, G4, L4, W5, G5, L5, G4b, L4b, G5b, L5b)

    pm = jnp.concatenate([pmU, pmI], axis=0)
    plv = jnp.concatenate([plvU, plvI], axis=0)
    return pm, pm, plv
